# baseline (speedup 1.0000x reference)
import functools

import jax
import jax.numpy as jnp
from jax import lax
from jax.experimental import pallas as pl
from jax.experimental.pallas import tpu as pltpu
from jax.experimental.pallas import tpu_sc as plsc

_DN = ('NHWC', 'HWIO', 'NHWC')


def _conv(x, w, b, s=1):
    return lax.conv_general_dilated(x, w, (s, s), 'SAME', dimension_numbers=_DN) + b


def _convT(x, w, b):
    outs = []
    for a in (0, 1):
        row = []
        for bb in (0, 1):
            wab = w[a::2, bb::2]
            c = lax.conv_general_dilated(
                x, wab, (1, 1), [(1 - a, a), (1 - bb, bb)],
                dimension_numbers=_DN)
            row.append(c)
        outs.append(jnp.stack(row, axis=3))
    t = jnp.stack(outs, axis=2)
    n, hh, _, ww, _, k = t.shape
    return t.reshape(n, hh * 2, ww * 2, k) + b


def _res_stack(x, w3, b3, w1, b1):
    h = _conv(jax.nn.relu(x), w3, b3)
    h = _conv(jax.nn.relu(h), w1, b1)
    return jax.nn.relu(x + h)


_BLK = 896


def _vq_idx_body(z_ref, cbt_ref, idx_ref):
    z = z_ref[...]
    cbt = cbt_ref[...]
    scores = jnp.dot(z, cbt, preferred_element_type=jnp.float32)
    cn = jnp.sum(cbt * cbt, axis=0, keepdims=True)
    zn = jnp.sum(z * z, axis=1, keepdims=True)
    d = (zn - 2.0 * scores) + cn
    dmin = jnp.min(d, axis=1, keepdims=True)
    k = d.shape[1]
    lane = lax.broadcasted_iota(jnp.int32, d.shape, 1)
    idx = jnp.min(jnp.where(d == dmin, lane, k), axis=1).astype(jnp.int32)
    idx_ref[...] = idx.reshape(1, 1, _BLK)


def _vq_indices(zf, cbt):
    n = zf.shape[0]
    grid = n // _BLK
    idx3 = pl.pallas_call(
        _vq_idx_body,
        grid=(grid,),
        in_specs=[
            pl.BlockSpec((_BLK, zf.shape[1]), lambda i: (i, 0)),
            pl.BlockSpec(cbt.shape, lambda i: (0, 0)),
        ],
        out_specs=pl.BlockSpec((1, 1, _BLK), lambda i: (i, 0, 0)),
        out_shape=jax.ShapeDtypeStruct((grid, 1, _BLK), jnp.int32),
    )(zf, cbt)
    return idx3.reshape(n)


_NC, _NS = 2, 16
_NW = _NC * _NS


def _sc_gather_body(bpw, dd, cb_hbm, idx_hbm, out_hbm, cb_v, idx_v, out_v,
                    sem):
    wid = lax.axis_index("s") * _NC + lax.axis_index("c")
    base = wid * bpw
    cb_cp = pltpu.async_copy(cb_hbm, cb_v, sem)
    pltpu.sync_copy(idx_hbm.at[pl.ds(base, bpw)], idx_v.at[pl.ds(0, bpw)])
    cb_cp.wait()

    lane = lax.iota(jnp.int32, 16)
    lane_d = lane * dd
    groups = bpw // 16
    rem = bpw - groups * 16

    @plsc.parallel_loop(0, groups, unroll=4)
    def _group(g):
        idx16 = idx_v[pl.ds(g * 16, 16)]
        src = idx16 * dd
        dstb = g * (16 * dd)
        for c in range(dd):
            v = plsc.load_gather(cb_v, [src + c])
            plsc.store_scatter(out_v, [lane_d + (dstb + c)], v)
    if rem:
        m = lane < rem
        idx16 = idx_v[pl.ds(groups * 16, 16)]
        src = idx16 * dd
        dstb = groups * (16 * dd)
        for c in range(dd):
            v = plsc.load_gather(cb_v, [src + c], mask=m)
            plsc.store_scatter(out_v, [lane_d + (dstb + c)], v, mask=m)
    pltpu.sync_copy(out_v, out_hbm.at[pl.ds(base * dd, bpw * dd)])


def _sc_gather(codebook, idx):
    n = idx.shape[0]
    k, dd = codebook.shape
    bpw = n // _NW
    mesh = plsc.VectorSubcoreMesh(core_axis_name="c", subcore_axis_name="s")
    run = pl.kernel(
        functools.partial(_sc_gather_body, bpw, dd),
        out_type=jax.ShapeDtypeStruct((n * dd,), jnp.float32),
        mesh=mesh,
        scratch_types=[
            pltpu.VMEM((k * dd,), jnp.float32),
            pltpu.VMEM((bpw + 16,), jnp.int32),
            pltpu.VMEM((bpw * dd,), jnp.float32),
            pltpu.SemaphoreType.DMA,
        ],
        compiler_params=pltpu.CompilerParams(use_tc_tiling_on_sc=False,
                                             needs_layout_passes=False),
    )
    return run(codebook.reshape(-1), idx).reshape(n, dd)


def kernel(x, we1, be1, we2, be2, we3, be3, wer3, ber3, wer1, ber1, wpre,
           bpre, codebook, wd1, bd1, wdr3, bdr3, wdr1, bdr1, wdt2, bdt2,
           wdtf, bdtf):
    h = jax.nn.relu(_conv(x, we1, be1, 2))
    h = jax.nn.relu(_conv(h, we2, be2, 2))
    h = jax.nn.relu(_conv(h, we3, be3, 1))
    h = _res_stack(h, wer3, ber3, wer1, ber1)

    z = _conv(h, wpre, bpre, 1)

    flat = z.reshape(-1, z.shape[-1])
    idx = _vq_indices(flat, codebook.T)
    q = _sc_gather(codebook, idx).reshape(z.shape)
    qst = z + lax.stop_gradient(q - z)

    y = _conv(qst, wd1, bd1, 1)
    y = _res_stack(y, wdr3, bdr3, wdr1, bdr1)
    y = jax.nn.relu(_convT(y, wdt2, bdt2))
    y = _convT(y, wdtf, bdtf)
    return y


# speedup vs baseline: 1.9864x; 1.9864x over previous
import functools

import jax
import jax.numpy as jnp
from jax import lax
from jax.experimental import pallas as pl
from jax.experimental.pallas import tpu as pltpu
from jax.experimental.pallas import tpu_sc as plsc

_DN = ('NHWC', 'HWIO', 'NHWC')


def _conv(x, w, b, s=1):
    return lax.conv_general_dilated(x, w, (s, s), 'SAME', dimension_numbers=_DN) + b


def _convT(x, w, b):
    return lax.conv_transpose(x, w, (2, 2), 'SAME', dimension_numbers=_DN) + b


def _res_stack(x, w3, b3, w1, b1):
    h = _conv(jax.nn.relu(x), w3, b3)
    h = _conv(jax.nn.relu(h), w1, b1)
    return jax.nn.relu(x + h)


_BLK = 896


def _vq_idx_body(z_ref, cbt_ref, idx_ref):
    z = z_ref[...]
    cbt = cbt_ref[...]
    scores = jnp.dot(z, cbt, preferred_element_type=jnp.float32)
    cn = jnp.sum(cbt * cbt, axis=0, keepdims=True)
    zn = jnp.sum(z * z, axis=1, keepdims=True)
    d = (zn - 2.0 * scores) + cn
    dmin = jnp.min(d, axis=1, keepdims=True)
    k = d.shape[1]
    lane = lax.broadcasted_iota(jnp.int32, d.shape, 1)
    idx = jnp.min(jnp.where(d == dmin, lane, k), axis=1).astype(jnp.int32)
    idx_ref[...] = idx.reshape(1, 1, _BLK)


def _vq_indices(zf, cbt):
    n = zf.shape[0]
    grid = n // _BLK
    idx3 = pl.pallas_call(
        _vq_idx_body,
        grid=(grid,),
        in_specs=[
            pl.BlockSpec((_BLK, zf.shape[1]), lambda i: (i, 0)),
            pl.BlockSpec(cbt.shape, lambda i: (0, 0)),
        ],
        out_specs=pl.BlockSpec((1, 1, _BLK), lambda i: (i, 0, 0)),
        out_shape=jax.ShapeDtypeStruct((grid, 1, _BLK), jnp.int32),
    )(zf, cbt)
    return idx3.reshape(n)


_NC, _NS = 2, 16
_NW = _NC * _NS


def _sc_gather_body(bpw, dd, cb_hbm, idx_hbm, out_hbm, cb_v, idx_v, out_v,
                    sem):
    wid = lax.axis_index("s") * _NC + lax.axis_index("c")
    base = wid * bpw
    cb_cp = pltpu.async_copy(cb_hbm, cb_v, sem)
    pltpu.sync_copy(idx_hbm.at[pl.ds(base, bpw)], idx_v.at[pl.ds(0, bpw)])
    cb_cp.wait()

    lane = lax.iota(jnp.int32, 16)
    lane_d = lane * dd
    groups = bpw // 16
    rem = bpw - groups * 16

    @plsc.parallel_loop(0, groups, unroll=8)
    def _group(g):
        idx16 = idx_v[pl.ds(g * 16, 16)]
        src = idx16 * dd
        dstb = g * (16 * dd)
        for c in range(dd):
            v = plsc.load_gather(cb_v, [src + c])
            plsc.store_scatter(out_v, [lane_d + (dstb + c)], v)
    if rem:
        m = lane < rem
        idx16 = idx_v[pl.ds(groups * 16, 16)]
        src = idx16 * dd
        dstb = groups * (16 * dd)
        for c in range(dd):
            v = plsc.load_gather(cb_v, [src + c], mask=m)
            plsc.store_scatter(out_v, [lane_d + (dstb + c)], v, mask=m)
    pltpu.sync_copy(out_v, out_hbm.at[pl.ds(base * dd, bpw * dd)])


def _sc_gather(codebook, idx):
    n = idx.shape[0]
    k, dd = codebook.shape
    bpw = n // _NW
    mesh = plsc.VectorSubcoreMesh(core_axis_name="c", subcore_axis_name="s")
    run = pl.kernel(
        functools.partial(_sc_gather_body, bpw, dd),
        out_type=jax.ShapeDtypeStruct((n * dd,), jnp.float32),
        mesh=mesh,
        scratch_types=[
            pltpu.VMEM((k * dd,), jnp.float32),
            pltpu.VMEM((bpw + 16,), jnp.int32),
            pltpu.VMEM((bpw * dd,), jnp.float32),
            pltpu.SemaphoreType.DMA,
        ],
        compiler_params=pltpu.CompilerParams(use_tc_tiling_on_sc=False,
                                             needs_layout_passes=False),
    )
    return run(codebook.reshape(-1), idx).reshape(n, dd)


def kernel(x, we1, be1, we2, be2, we3, be3, wer3, ber3, wer1, ber1, wpre,
           bpre, codebook, wd1, bd1, wdr3, bdr3, wdr1, bdr1, wdt2, bdt2,
           wdtf, bdtf):
    h = jax.nn.relu(_conv(x, we1, be1, 2))
    h = jax.nn.relu(_conv(h, we2, be2, 2))
    h = jax.nn.relu(_conv(h, we3, be3, 1))
    h = _res_stack(h, wer3, ber3, wer1, ber1)

    z = _conv(h, wpre, bpre, 1)

    flat = z.reshape(-1, z.shape[-1])
    idx = _vq_indices(flat, codebook.T)
    q = _sc_gather(codebook, idx).reshape(z.shape)
    qst = z + lax.stop_gradient(q - z)

    y = _conv(qst, wd1, bd1, 1)
    y = _res_stack(y, wdr3, bdr3, wdr1, bdr1)
    y = jax.nn.relu(_convT(y, wdt2, bdt2))
    y = _convT(y, wdtf, bdtf)
    return y


# speedup vs baseline: 1.9884x; 1.0010x over previous
import functools

import jax
import jax.numpy as jnp
from jax import lax
from jax.experimental import pallas as pl
from jax.experimental.pallas import tpu as pltpu
from jax.experimental.pallas import tpu_sc as plsc

_DN = ('NHWC', 'HWIO', 'NHWC')


def _conv(x, w, b, s=1):
    return lax.conv_general_dilated(x, w, (s, s), 'SAME', dimension_numbers=_DN) + b


def _convT(x, w, b):
    return lax.conv_transpose(x, w, (2, 2), 'SAME', dimension_numbers=_DN) + b


def _res_stack(x, w3, b3, w1, b1):
    h = _conv(jax.nn.relu(x), w3, b3)
    h = _conv(jax.nn.relu(h), w1, b1)
    return jax.nn.relu(x + h)


_BLK = 896


def _vq_idx_body(z_ref, cbt_ref, idx_ref):
    z = z_ref[...]
    cbt = cbt_ref[...]
    scores = jnp.dot(z, cbt, preferred_element_type=jnp.float32)
    cn = jnp.sum(cbt * cbt, axis=0, keepdims=True)
    zn = jnp.sum(z * z, axis=1, keepdims=True)
    d = (zn - 2.0 * scores) + cn
    idx = jnp.argmin(d, axis=1).astype(jnp.int32)
    idx_ref[...] = idx.reshape(1, 1, _BLK)


def _vq_indices(zf, cbt):
    n = zf.shape[0]
    grid = n // _BLK
    idx3 = pl.pallas_call(
        _vq_idx_body,
        grid=(grid,),
        in_specs=[
            pl.BlockSpec((_BLK, zf.shape[1]), lambda i: (i, 0)),
            pl.BlockSpec(cbt.shape, lambda i: (0, 0)),
        ],
        out_specs=pl.BlockSpec((1, 1, _BLK), lambda i: (i, 0, 0)),
        out_shape=jax.ShapeDtypeStruct((grid, 1, _BLK), jnp.int32),
    )(zf, cbt)
    return idx3.reshape(n)


_NC, _NS = 2, 16
_NW = _NC * _NS


def _sc_gather_body(bpw, dd, cb_hbm, idx_hbm, out_hbm, cb_v, idx_v, out_v,
                    sem):
    wid = lax.axis_index("s") * _NC + lax.axis_index("c")
    base = wid * bpw
    cb_cp = pltpu.async_copy(cb_hbm, cb_v, sem)
    pltpu.sync_copy(idx_hbm.at[pl.ds(base, bpw)], idx_v.at[pl.ds(0, bpw)])
    cb_cp.wait()

    lane = lax.iota(jnp.int32, 16)
    lane_d = lane * dd
    groups = bpw // 16
    rem = bpw - groups * 16

    @plsc.parallel_loop(0, groups, unroll=4)
    def _group(g):
        idx16 = idx_v[pl.ds(g * 16, 16)]
        src = idx16 * dd
        dstb = g * (16 * dd)
        for c in range(dd):
            v = plsc.load_gather(cb_v, [src + c])
            plsc.store_scatter(out_v, [lane_d + (dstb + c)], v)
    if rem:
        m = lane < rem
        idx16 = idx_v[pl.ds(groups * 16, 16)]
        src = idx16 * dd
        dstb = groups * (16 * dd)
        for c in range(dd):
            v = plsc.load_gather(cb_v, [src + c], mask=m)
            plsc.store_scatter(out_v, [lane_d + (dstb + c)], v, mask=m)
    pltpu.sync_copy(out_v, out_hbm.at[pl.ds(base * dd, bpw * dd)])


def _sc_gather(codebook, idx):
    n = idx.shape[0]
    k, dd = codebook.shape
    bpw = n // _NW
    mesh = plsc.VectorSubcoreMesh(core_axis_name="c", subcore_axis_name="s")
    run = pl.kernel(
        functools.partial(_sc_gather_body, bpw, dd),
        out_type=jax.ShapeDtypeStruct((n * dd,), jnp.float32),
        mesh=mesh,
        scratch_types=[
            pltpu.VMEM((k * dd,), jnp.float32),
            pltpu.VMEM((bpw + 16,), jnp.int32),
            pltpu.VMEM((bpw * dd,), jnp.float32),
            pltpu.SemaphoreType.DMA,
        ],
        compiler_params=pltpu.CompilerParams(use_tc_tiling_on_sc=False,
                                             needs_layout_passes=False),
    )
    return run(codebook.reshape(-1), idx).reshape(n, dd)


def kernel(x, we1, be1, we2, be2, we3, be3, wer3, ber3, wer1, ber1, wpre,
           bpre, codebook, wd1, bd1, wdr3, bdr3, wdr1, bdr1, wdt2, bdt2,
           wdtf, bdtf):
    h = jax.nn.relu(_conv(x, we1, be1, 2))
    h = jax.nn.relu(_conv(h, we2, be2, 2))
    h = jax.nn.relu(_conv(h, we3, be3, 1))
    h = _res_stack(h, wer3, ber3, wer1, ber1)

    z = _conv(h, wpre, bpre, 1)

    flat = z.reshape(-1, z.shape[-1])
    idx = _vq_indices(flat, codebook.T)
    q = _sc_gather(codebook, idx).reshape(z.shape)
    qst = z + lax.stop_gradient(q - z)

    y = _conv(qst, wd1, bd1, 1)
    y = _res_stack(y, wdr3, bdr3, wdr1, bdr1)
    y = jax.nn.relu(_convT(y, wdt2, bdt2))
    y = _convT(y, wdtf, bdtf)
    return y


# speedup vs baseline: 1.9976x; 1.0046x over previous
import functools

import jax
import jax.numpy as jnp
from jax import lax
from jax.experimental import pallas as pl
from jax.experimental.pallas import tpu as pltpu
from jax.experimental.pallas import tpu_sc as plsc

_DN = ('NHWC', 'HWIO', 'NHWC')


def _conv(x, w, b, s=1):
    return lax.conv_general_dilated(x, w, (s, s), 'SAME', dimension_numbers=_DN) + b


def _convT(x, w, b):
    return lax.conv_transpose(x, w, (2, 2), 'SAME', dimension_numbers=_DN) + b


def _res_stack(x, w3, b3, w1, b1):
    h = _conv(jax.nn.relu(x), w3, b3)
    h = _conv(jax.nn.relu(h), w1, b1)
    return jax.nn.relu(x + h)


_BLK = 1792


def _vq_idx_body(z_ref, cbt_ref, idx_ref):
    z = z_ref[...]
    cbt = cbt_ref[...]
    scores = jnp.dot(z, cbt, preferred_element_type=jnp.float32)
    cn = jnp.sum(cbt * cbt, axis=0, keepdims=True)
    zn = jnp.sum(z * z, axis=1, keepdims=True)
    d = (zn - 2.0 * scores) + cn
    idx = jnp.argmin(d, axis=1).astype(jnp.int32)
    idx_ref[...] = idx.reshape(1, 1, _BLK)


def _vq_indices(zf, cbt):
    n = zf.shape[0]
    grid = n // _BLK
    idx3 = pl.pallas_call(
        _vq_idx_body,
        grid=(grid,),
        in_specs=[
            pl.BlockSpec((_BLK, zf.shape[1]), lambda i: (i, 0)),
            pl.BlockSpec(cbt.shape, lambda i: (0, 0)),
        ],
        out_specs=pl.BlockSpec((1, 1, _BLK), lambda i: (i, 0, 0)),
        out_shape=jax.ShapeDtypeStruct((grid, 1, _BLK), jnp.int32),
    )(zf, cbt)
    return idx3.reshape(n)


_NC, _NS = 2, 16
_NW = _NC * _NS


def _sc_gather_body(bpw, dd, cb_hbm, idx_hbm, out_hbm, cb_v, idx_v, out_v,
                    sem):
    wid = lax.axis_index("s") * _NC + lax.axis_index("c")
    base = wid * bpw
    cb_cp = pltpu.async_copy(cb_hbm, cb_v, sem)
    pltpu.sync_copy(idx_hbm.at[pl.ds(base, bpw)], idx_v.at[pl.ds(0, bpw)])
    cb_cp.wait()

    lane = lax.iota(jnp.int32, 16)
    lane_d = lane * dd
    groups = bpw // 16
    rem = bpw - groups * 16

    @plsc.parallel_loop(0, groups, unroll=4)
    def _group(g):
        idx16 = idx_v[pl.ds(g * 16, 16)]
        src = idx16 * dd
        dstb = g * (16 * dd)
        for c in range(dd):
            v = plsc.load_gather(cb_v, [src + c])
            plsc.store_scatter(out_v, [lane_d + (dstb + c)], v)
    if rem:
        m = lane < rem
        idx16 = idx_v[pl.ds(groups * 16, 16)]
        src = idx16 * dd
        dstb = groups * (16 * dd)
        for c in range(dd):
            v = plsc.load_gather(cb_v, [src + c], mask=m)
            plsc.store_scatter(out_v, [lane_d + (dstb + c)], v, mask=m)
    pltpu.sync_copy(out_v, out_hbm.at[pl.ds(base * dd, bpw * dd)])


def _sc_gather(codebook, idx):
    n = idx.shape[0]
    k, dd = codebook.shape
    bpw = n // _NW
    mesh = plsc.VectorSubcoreMesh(core_axis_name="c", subcore_axis_name="s")
    run = pl.kernel(
        functools.partial(_sc_gather_body, bpw, dd),
        out_type=jax.ShapeDtypeStruct((n * dd,), jnp.float32),
        mesh=mesh,
        scratch_types=[
            pltpu.VMEM((k * dd,), jnp.float32),
            pltpu.VMEM((bpw + 16,), jnp.int32),
            pltpu.VMEM((bpw * dd,), jnp.float32),
            pltpu.SemaphoreType.DMA,
        ],
        compiler_params=pltpu.CompilerParams(use_tc_tiling_on_sc=False,
                                             needs_layout_passes=False),
    )
    return run(codebook.reshape(-1), idx).reshape(n, dd)


def kernel(x, we1, be1, we2, be2, we3, be3, wer3, ber3, wer1, ber1, wpre,
           bpre, codebook, wd1, bd1, wdr3, bdr3, wdr1, bdr1, wdt2, bdt2,
           wdtf, bdtf):
    h = jax.nn.relu(_conv(x, we1, be1, 2))
    h = jax.nn.relu(_conv(h, we2, be2, 2))
    h = jax.nn.relu(_conv(h, we3, be3, 1))
    h = _res_stack(h, wer3, ber3, wer1, ber1)

    z = _conv(h, wpre, bpre, 1)

    flat = z.reshape(-1, z.shape[-1])
    idx = _vq_indices(flat, codebook.T)
    q = _sc_gather(codebook, idx).reshape(z.shape)
    qst = z + lax.stop_gradient(q - z)

    y = _conv(qst, wd1, bd1, 1)
    y = _res_stack(y, wdr3, bdr3, wdr1, bdr1)
    y = jax.nn.relu(_convT(y, wdt2, bdt2))
    y = _convT(y, wdtf, bdtf)
    return y
